# Initial kernel scaffold; baseline (speedup 1.0000x reference)
import jax
import jax.numpy as jnp
from jax.experimental import pallas as pl


def kernel(inputs):
    raise NotImplementedError("write your pallas kernel here")


# speedup vs baseline: 183.4649x; 183.4649x over previous
import functools

import jax
import jax.numpy as jnp
from jax import lax
from jax.experimental import pallas as pl
from jax.experimental.pallas import tpu as pltpu
from jax.experimental.pallas import tpu_sc as plsc

NC = 2
NS = 16
L = 16
NW = NC * NS

TOTAL = 2048 * 8192
PER_W = TOTAL // NW
CHUNK = 16384
NSTEPS = PER_W // CHUNK
ELEMS_PER_IT = 4 * L
NIT = CHUNK // ELEMS_PER_IT

_mesh = plsc.VectorSubcoreMesh(
    core_axis_name="c", subcore_axis_name="s", num_cores=NC, num_subcores=NS
)


def _maj2(a, b, c):
    return (a & b) | (c & (a | b))


@functools.partial(
    pl.kernel,
    out_type=jax.ShapeDtypeStruct((TOTAL,), jnp.float32),
    mesh=_mesh,
    scratch_types=[
        pltpu.VMEM((CHUNK,), jnp.float32),
    ],
    compiler_params=pltpu.CompilerParams(needs_layout_passes=False),
)
def _sparsity_sc(x_hbm, out_hbm, buf):
    wid = lax.axis_index("s") * NC + lax.axis_index("c")
    base = wid * PER_W
    iota4 = lax.iota(jnp.int32, L) * 4
    zero = jnp.zeros((L,), jnp.float32)

    def step(g, carry):
        off = base + g * CHUNK
        pltpu.sync_copy(x_hbm.at[pl.ds(off, CHUNK)], buf)

        def inner(i, carry2):
            idx0 = iota4 + i * ELEMS_PER_IT
            idx1 = idx0 + 1
            idx2 = idx0 + 2
            idx3 = idx0 + 3
            x0 = plsc.load_gather(buf, [idx0])
            x1 = plsc.load_gather(buf, [idx1])
            x2 = plsc.load_gather(buf, [idx2])
            x3 = plsc.load_gather(buf, [idx3])
            a0 = jnp.abs(x0)
            a1 = jnp.abs(x1)
            a2 = jnp.abs(x2)
            a3 = jnp.abs(x3)
            c01 = a0 > a1
            c02 = a0 > a2
            c03 = a0 > a3
            c12 = a1 > a2
            c13 = a1 > a3
            c23 = a2 > a3
            n01 = ~c01
            n02 = ~c02
            n03 = ~c03
            n12 = ~c12
            n13 = ~c13
            n23 = ~c23
            k0 = _maj2(c01, c02, c03)
            k1 = _maj2(n01, c12, c13)
            k2 = _maj2(n02, n12, c23)
            k3 = _maj2(n03, n13, n23)
            plsc.store_scatter(buf, [idx0], jnp.where(k0, x0, zero))
            plsc.store_scatter(buf, [idx1], jnp.where(k1, x1, zero))
            plsc.store_scatter(buf, [idx2], jnp.where(k2, x2, zero))
            plsc.store_scatter(buf, [idx3], jnp.where(k3, x3, zero))
            return carry2

        lax.fori_loop(0, NIT, inner, 0, unroll=2)
        pltpu.sync_copy(buf, out_hbm.at[pl.ds(off, CHUNK)])
        return carry

    lax.fori_loop(0, NSTEPS, step, 0)


def kernel(inputs):
    flat = inputs.reshape(TOTAL)
    out = _sparsity_sc(flat)
    return out.reshape(inputs.shape)


# speedup vs baseline: 234.3998x; 1.2776x over previous
import functools

import jax
import jax.numpy as jnp
from jax import lax
from jax.experimental import pallas as pl
from jax.experimental.pallas import tpu as pltpu
from jax.experimental.pallas import tpu_sc as plsc

NC = 2
NS = 16
L = 16
NW = NC * NS

TOTAL = 2048 * 8192
PER_W = TOTAL // NW
CHUNK = 16384
NSTEPS = PER_W // CHUNK
ELEMS_PER_IT = 4 * L
NIT = CHUNK // ELEMS_PER_IT

_mesh = plsc.VectorSubcoreMesh(
    core_axis_name="c", subcore_axis_name="s", num_cores=NC, num_subcores=NS
)


def _maj2(a, b, c):
    return (a & b) | (c & (a | b))


@functools.partial(
    pl.kernel,
    out_type=jax.ShapeDtypeStruct((TOTAL,), jnp.float32),
    mesh=_mesh,
    scratch_types=[
        pltpu.VMEM((CHUNK,), jnp.float32),
        pltpu.VMEM((CHUNK,), jnp.float32),
        pltpu.VMEM((CHUNK,), jnp.float32),
        pltpu.VMEM((CHUNK,), jnp.float32),
        pltpu.SemaphoreType.DMA,
        pltpu.SemaphoreType.DMA,
        pltpu.SemaphoreType.DMA,
        pltpu.SemaphoreType.DMA,
    ],
    compiler_params=pltpu.CompilerParams(needs_layout_passes=False),
)
def _sparsity_sc(x_hbm, out_hbm, in0, in1, out0, out1,
                 sem_i0, sem_i1, sem_o0, sem_o1):
    wid = lax.axis_index("s") * NC + lax.axis_index("c")
    base = wid * PER_W
    iota4 = lax.iota(jnp.int32, L) * 4
    zero = jnp.zeros((L,), jnp.float32)

    def compute(src, dst):
        def inner(i, idx0):
            idx1 = idx0 + 1
            idx2 = idx0 + 2
            idx3 = idx0 + 3
            x0 = plsc.load_gather(src, [idx0])
            x1 = plsc.load_gather(src, [idx1])
            x2 = plsc.load_gather(src, [idx2])
            x3 = plsc.load_gather(src, [idx3])
            a0 = jnp.abs(x0)
            a1 = jnp.abs(x1)
            a2 = jnp.abs(x2)
            a3 = jnp.abs(x3)
            c01 = a0 > a1
            c02 = a0 > a2
            c03 = a0 > a3
            c12 = a1 > a2
            c13 = a1 > a3
            c23 = a2 > a3
            k0 = _maj2(c01, c02, c03)
            k1 = _maj2(~c01, c12, c13)
            k2 = _maj2(~c02, ~c12, c23)
            k3 = ~_maj2(c03, c13, c23)
            plsc.store_scatter(dst, [idx0], jnp.where(k0, x0, zero))
            plsc.store_scatter(dst, [idx1], jnp.where(k1, x1, zero))
            plsc.store_scatter(dst, [idx2], jnp.where(k2, x2, zero))
            plsc.store_scatter(dst, [idx3], jnp.where(k3, x3, zero))
            return idx0 + ELEMS_PER_IT

        lax.fori_loop(0, NIT, inner, iota4, unroll=4)

    def halfstep(g, inb, outb, sem_i, sem_o):
        off = base + g * CHUNK
        pltpu.make_async_copy(x_hbm.at[pl.ds(off, CHUNK)], inb, sem_i).wait()
        @pl.when(g >= 2)
        def _():
            pltpu.make_async_copy(
                outb, out_hbm.at[pl.ds(off, CHUNK)], sem_o).wait()

        compute(inb, outb)
        pltpu.async_copy(outb, out_hbm.at[pl.ds(off, CHUNK)], sem_o)

        @pl.when(g + 2 < NSTEPS)
        def _():
            off2 = off + 2 * CHUNK
            pltpu.async_copy(x_hbm.at[pl.ds(off2, CHUNK)], inb, sem_i)

    pltpu.async_copy(x_hbm.at[pl.ds(base, CHUNK)], in0, sem_i0)
    pltpu.async_copy(x_hbm.at[pl.ds(base + CHUNK, CHUNK)], in1, sem_i1)

    def step(gg, carry):
        g = gg * 2
        halfstep(g, in0, out0, sem_i0, sem_o0)
        halfstep(g + 1, in1, out1, sem_i1, sem_o1)
        return carry

    lax.fori_loop(0, NSTEPS // 2, step, 0)

    pltpu.make_async_copy(
        out0, out_hbm.at[pl.ds(base, CHUNK)], sem_o0).wait()
    pltpu.make_async_copy(
        out1, out_hbm.at[pl.ds(base, CHUNK)], sem_o1).wait()


def kernel(inputs):
    flat = inputs.reshape(TOTAL)
    out = _sparsity_sc(flat)
    return out.reshape(inputs.shape)


# speedup vs baseline: 236.7196x; 1.0099x over previous
import functools

import jax
import jax.numpy as jnp
from jax import lax
from jax.experimental import pallas as pl
from jax.experimental.pallas import tpu as pltpu
from jax.experimental.pallas import tpu_sc as plsc

NC = 2
NS = 16
L = 16
NW = NC * NS

TOTAL = 2048 * 8192
PER_W = TOTAL // NW
CHUNK = 16384
NSTEPS = PER_W // CHUNK
ELEMS_PER_IT = 4 * L
NIT = CHUNK // ELEMS_PER_IT

_mesh = plsc.VectorSubcoreMesh(
    core_axis_name="c", subcore_axis_name="s", num_cores=NC, num_subcores=NS
)


def _maj2(a, b, c):
    return (a & b) | (c & (a | b))


@functools.partial(
    pl.kernel,
    out_type=jax.ShapeDtypeStruct((TOTAL,), jnp.float32),
    mesh=_mesh,
    scratch_types=[
        pltpu.VMEM((CHUNK,), jnp.float32),
        pltpu.VMEM((CHUNK,), jnp.float32),
        pltpu.VMEM((CHUNK,), jnp.float32),
        pltpu.VMEM((CHUNK,), jnp.float32),
        pltpu.SemaphoreType.DMA,
        pltpu.SemaphoreType.DMA,
        pltpu.SemaphoreType.DMA,
        pltpu.SemaphoreType.DMA,
    ],
    compiler_params=pltpu.CompilerParams(needs_layout_passes=False),
)
def _sparsity_sc(x_hbm, out_hbm, in0, in1, out0, out1,
                 sem_i0, sem_i1, sem_o0, sem_o1):
    wid = lax.axis_index("s") * NC + lax.axis_index("c")
    base = wid * PER_W
    iota4 = lax.iota(jnp.int32, L) * 4
    zero = jnp.zeros((L,), jnp.float32)

    def compute(src, dst):
        def inner(i, idx0):
            idx1 = idx0 + 1
            idx2 = idx0 + 2
            idx3 = idx0 + 3
            x0 = plsc.load_gather(src, [idx0])
            x1 = plsc.load_gather(src, [idx1])
            x2 = plsc.load_gather(src, [idx2])
            x3 = plsc.load_gather(src, [idx3])
            a0 = jnp.abs(x0)
            a1 = jnp.abs(x1)
            a2 = jnp.abs(x2)
            a3 = jnp.abs(x3)
            c01 = a0 > a1
            c02 = a0 > a2
            c03 = a0 > a3
            c12 = a1 > a2
            c13 = a1 > a3
            c23 = a2 > a3
            k0 = _maj2(c01, c02, c03)
            k1 = _maj2(~c01, c12, c13)
            k2 = _maj2(~c02, ~c12, c23)
            k3 = k0 ^ k1 ^ k2
            plsc.store_scatter(dst, [idx0], jnp.where(k0, x0, zero))
            plsc.store_scatter(dst, [idx1], jnp.where(k1, x1, zero))
            plsc.store_scatter(dst, [idx2], jnp.where(k2, x2, zero))
            plsc.store_scatter(dst, [idx3], jnp.where(k3, x3, zero))
            return idx0 + ELEMS_PER_IT

        lax.fori_loop(0, NIT, inner, iota4, unroll=8)

    def halfstep(g, inb, outb, sem_i, sem_o):
        off = base + g * CHUNK
        pltpu.make_async_copy(x_hbm.at[pl.ds(off, CHUNK)], inb, sem_i).wait()
        @pl.when(g >= 2)
        def _():
            pltpu.make_async_copy(
                outb, out_hbm.at[pl.ds(off, CHUNK)], sem_o).wait()

        compute(inb, outb)
        pltpu.async_copy(outb, out_hbm.at[pl.ds(off, CHUNK)], sem_o)

        @pl.when(g + 2 < NSTEPS)
        def _():
            off2 = off + 2 * CHUNK
            pltpu.async_copy(x_hbm.at[pl.ds(off2, CHUNK)], inb, sem_i)

    pltpu.async_copy(x_hbm.at[pl.ds(base, CHUNK)], in0, sem_i0)
    pltpu.async_copy(x_hbm.at[pl.ds(base + CHUNK, CHUNK)], in1, sem_i1)

    def step(gg, carry):
        g = gg * 2
        halfstep(g, in0, out0, sem_i0, sem_o0)
        halfstep(g + 1, in1, out1, sem_i1, sem_o1)
        return carry

    lax.fori_loop(0, NSTEPS // 2, step, 0)

    pltpu.make_async_copy(
        out0, out_hbm.at[pl.ds(base, CHUNK)], sem_o0).wait()
    pltpu.make_async_copy(
        out1, out_hbm.at[pl.ds(base, CHUNK)], sem_o1).wait()


def kernel(inputs):
    flat = inputs.reshape(TOTAL)
    out = _sparsity_sc(flat)
    return out.reshape(inputs.shape)


# speedup vs baseline: 283.3400x; 1.1969x over previous
import functools

import jax
import jax.numpy as jnp
from jax import lax
from jax.experimental import pallas as pl
from jax.experimental.pallas import tpu as pltpu
from jax.experimental.pallas import tpu_sc as plsc

NC = 2
NS = 16
L = 16
NW = NC * NS

TOTAL = 2048 * 8192
PER_W = TOTAL // NW
CHUNK = 16384
NSTEPS = PER_W // CHUNK
ELEMS_PER_IT = 4 * L
NIT = CHUNK // ELEMS_PER_IT

_mesh = plsc.VectorSubcoreMesh(
    core_axis_name="c", subcore_axis_name="s", num_cores=NC, num_subcores=NS
)


def _sign(d):
    return lax.shift_right_logical(
        lax.bitcast_convert_type(d, jnp.int32), jnp.int32(31))


@functools.partial(
    pl.kernel,
    out_type=jax.ShapeDtypeStruct((TOTAL,), jnp.float32),
    mesh=_mesh,
    scratch_types=[
        pltpu.VMEM((CHUNK,), jnp.float32),
        pltpu.VMEM((CHUNK,), jnp.float32),
        pltpu.VMEM((CHUNK,), jnp.float32),
        pltpu.VMEM((CHUNK,), jnp.float32),
        pltpu.SemaphoreType.DMA,
        pltpu.SemaphoreType.DMA,
        pltpu.SemaphoreType.DMA,
        pltpu.SemaphoreType.DMA,
    ],
    compiler_params=pltpu.CompilerParams(needs_layout_passes=False),
)
def _sparsity_sc(x_hbm, out_hbm, in0, in1, out0, out1,
                 sem_i0, sem_i1, sem_o0, sem_o1):
    wid = lax.axis_index("s") * NC + lax.axis_index("c")
    base = wid * PER_W
    iota4 = lax.iota(jnp.int32, L) * 4
    zero = jnp.zeros((L,), jnp.float32)

    def compute(src, dst):
        @plsc.parallel_loop(0, NIT, step=1, unroll=4)
        def _body(i):
            idx0 = iota4 + i * ELEMS_PER_IT
            idx1 = idx0 + 1
            idx2 = idx0 + 2
            idx3 = idx0 + 3
            x0 = plsc.load_gather(src, [idx0])
            x1 = plsc.load_gather(src, [idx1])
            x2 = plsc.load_gather(src, [idx2])
            x3 = plsc.load_gather(src, [idx3])
            a0 = jnp.abs(x0)
            a1 = jnp.abs(x1)
            a2 = jnp.abs(x2)
            a3 = jnp.abs(x3)
            s01 = _sign(a1 - a0)
            s02 = _sign(a2 - a0)
            s03 = _sign(a3 - a0)
            s12 = _sign(a2 - a1)
            s13 = _sign(a3 - a1)
            s23 = _sign(a3 - a2)
            k0 = (s01 + s02) + s03 >= 2
            k1 = (s12 + s13) - s01 >= 1
            k2 = s23 - (s02 + s12) >= 0
            k3 = (s03 + s13) + s23 <= 1
            plsc.store_scatter(dst, [idx0], jnp.where(k0, x0, zero))
            plsc.store_scatter(dst, [idx1], jnp.where(k1, x1, zero))
            plsc.store_scatter(dst, [idx2], jnp.where(k2, x2, zero))
            plsc.store_scatter(dst, [idx3], jnp.where(k3, x3, zero))

    def halfstep(g, inb, outb, sem_i, sem_o):
        off = base + g * CHUNK
        pltpu.make_async_copy(x_hbm.at[pl.ds(off, CHUNK)], inb, sem_i).wait()
        @pl.when(g >= 2)
        def _():
            pltpu.make_async_copy(
                outb, out_hbm.at[pl.ds(off, CHUNK)], sem_o).wait()

        compute(inb, outb)
        pltpu.async_copy(outb, out_hbm.at[pl.ds(off, CHUNK)], sem_o)

        @pl.when(g + 2 < NSTEPS)
        def _():
            off2 = off + 2 * CHUNK
            pltpu.async_copy(x_hbm.at[pl.ds(off2, CHUNK)], inb, sem_i)

    pltpu.async_copy(x_hbm.at[pl.ds(base, CHUNK)], in0, sem_i0)
    pltpu.async_copy(x_hbm.at[pl.ds(base + CHUNK, CHUNK)], in1, sem_i1)

    def step(gg, carry):
        g = gg * 2
        halfstep(g, in0, out0, sem_i0, sem_o0)
        halfstep(g + 1, in1, out1, sem_i1, sem_o1)
        return carry

    lax.fori_loop(0, NSTEPS // 2, step, 0)

    pltpu.make_async_copy(
        out0, out_hbm.at[pl.ds(base, CHUNK)], sem_o0).wait()
    pltpu.make_async_copy(
        out1, out_hbm.at[pl.ds(base, CHUNK)], sem_o1).wait()


def kernel(inputs):
    flat = inputs.reshape(TOTAL)
    out = _sparsity_sc(flat)
    return out.reshape(inputs.shape)


# speedup vs baseline: 293.5428x; 1.0360x over previous
import functools

import jax
import jax.numpy as jnp
from jax import lax
from jax.experimental import pallas as pl
from jax.experimental.pallas import tpu as pltpu
from jax.experimental.pallas import tpu_sc as plsc

NC = 2
NS = 16
L = 16
NW = NC * NS

TOTAL = 2048 * 8192
PER_W = TOTAL // NW
CHUNK = 16384
NSTEPS = PER_W // CHUNK
ELEMS_PER_IT = 4 * L
NIT = CHUNK // ELEMS_PER_IT

_mesh = plsc.VectorSubcoreMesh(
    core_axis_name="c", subcore_axis_name="s", num_cores=NC, num_subcores=NS
)


def _sign(d):
    return lax.shift_right_logical(
        lax.bitcast_convert_type(d, jnp.int32), jnp.int32(31))


@functools.partial(
    pl.kernel,
    out_type=jax.ShapeDtypeStruct((TOTAL,), jnp.float32),
    mesh=_mesh,
    scratch_types=[
        pltpu.VMEM((CHUNK,), jnp.float32),
        pltpu.VMEM((CHUNK,), jnp.float32),
        pltpu.VMEM((CHUNK,), jnp.float32),
        pltpu.VMEM((CHUNK,), jnp.float32),
        pltpu.SemaphoreType.DMA,
        pltpu.SemaphoreType.DMA,
        pltpu.SemaphoreType.DMA,
        pltpu.SemaphoreType.DMA,
    ],
    compiler_params=pltpu.CompilerParams(needs_layout_passes=False),
)
def _sparsity_sc(x_hbm, out_hbm, in0, in1, out0, out1,
                 sem_i0, sem_i1, sem_o0, sem_o1):
    wid = lax.axis_index("s") * NC + lax.axis_index("c")
    base = wid * PER_W
    iota4 = lax.iota(jnp.int32, L) * 4
    idx0 = iota4
    idx1 = iota4 + 1
    idx2 = iota4 + 2
    idx3 = iota4 + 3
    zero = jnp.zeros((L,), jnp.float32)

    def compute(src, dst):
        @plsc.parallel_loop(0, NIT, step=1, unroll=4)
        def _body(i):
            b = i * ELEMS_PER_IT
            s = src.at[pl.ds(b, ELEMS_PER_IT)]
            d = dst.at[pl.ds(b, ELEMS_PER_IT)]
            x0 = plsc.load_gather(s, [idx0])
            x1 = plsc.load_gather(s, [idx1])
            x2 = plsc.load_gather(s, [idx2])
            x3 = plsc.load_gather(s, [idx3])
            a0 = jnp.abs(x0)
            a1 = jnp.abs(x1)
            a2 = jnp.abs(x2)
            a3 = jnp.abs(x3)
            s01 = _sign(a1 - a0)
            s02 = _sign(a2 - a0)
            s03 = _sign(a3 - a0)
            s12 = _sign(a2 - a1)
            s13 = _sign(a3 - a1)
            s23 = _sign(a3 - a2)
            k0 = (s01 + s02) + s03 >= 2
            k1 = (s12 + s13) - s01 >= 1
            k2 = s23 - (s02 + s12) >= 0
            k3 = (s03 + s13) + s23 <= 1
            plsc.store_scatter(d, [idx0], jnp.where(k0, x0, zero))
            plsc.store_scatter(d, [idx1], jnp.where(k1, x1, zero))
            plsc.store_scatter(d, [idx2], jnp.where(k2, x2, zero))
            plsc.store_scatter(d, [idx3], jnp.where(k3, x3, zero))

    def halfstep(g, inb, outb, sem_i, sem_o):
        off = base + g * CHUNK
        pltpu.make_async_copy(x_hbm.at[pl.ds(off, CHUNK)], inb, sem_i).wait()
        @pl.when(g >= 2)
        def _():
            pltpu.make_async_copy(
                outb, out_hbm.at[pl.ds(off, CHUNK)], sem_o).wait()

        compute(inb, outb)
        pltpu.async_copy(outb, out_hbm.at[pl.ds(off, CHUNK)], sem_o)

        @pl.when(g + 2 < NSTEPS)
        def _():
            off2 = off + 2 * CHUNK
            pltpu.async_copy(x_hbm.at[pl.ds(off2, CHUNK)], inb, sem_i)

    pltpu.async_copy(x_hbm.at[pl.ds(base, CHUNK)], in0, sem_i0)
    pltpu.async_copy(x_hbm.at[pl.ds(base + CHUNK, CHUNK)], in1, sem_i1)

    def step(gg, carry):
        g = gg * 2
        halfstep(g, in0, out0, sem_i0, sem_o0)
        halfstep(g + 1, in1, out1, sem_i1, sem_o1)
        return carry

    lax.fori_loop(0, NSTEPS // 2, step, 0)

    pltpu.make_async_copy(
        out0, out_hbm.at[pl.ds(base, CHUNK)], sem_o0).wait()
    pltpu.make_async_copy(
        out1, out_hbm.at[pl.ds(base, CHUNK)], sem_o1).wait()


def kernel(inputs):
    flat = inputs.reshape(TOTAL)
    out = _sparsity_sc(flat)
    return out.reshape(inputs.shape)


# speedup vs baseline: 299.9505x; 1.0218x over previous
import functools

import jax
import jax.numpy as jnp
from jax import lax
from jax.experimental import pallas as pl
from jax.experimental.pallas import tpu as pltpu
from jax.experimental.pallas import tpu_sc as plsc

NC = 2
NS = 16
L = 16
NW = NC * NS

TOTAL = 2048 * 8192
PER_W = TOTAL // NW
CHUNK = 16384
NSTEPS = PER_W // CHUNK
ELEMS_PER_IT = 4 * L
NIT = CHUNK // ELEMS_PER_IT

_mesh = plsc.VectorSubcoreMesh(
    core_axis_name="c", subcore_axis_name="s", num_cores=NC, num_subcores=NS
)


def _sign(d):
    return lax.shift_right_logical(
        lax.bitcast_convert_type(d, jnp.int32), jnp.int32(31))


@functools.partial(
    pl.kernel,
    out_type=jax.ShapeDtypeStruct((TOTAL,), jnp.float32),
    mesh=_mesh,
    scratch_types=[
        pltpu.VMEM((CHUNK,), jnp.float32),
        pltpu.VMEM((CHUNK,), jnp.float32),
        pltpu.VMEM((CHUNK,), jnp.float32),
        pltpu.VMEM((CHUNK,), jnp.float32),
        pltpu.SemaphoreType.DMA,
        pltpu.SemaphoreType.DMA,
        pltpu.SemaphoreType.DMA,
        pltpu.SemaphoreType.DMA,
    ],
    compiler_params=pltpu.CompilerParams(needs_layout_passes=False),
)
def _sparsity_sc(x_hbm, out_hbm, in0, in1, out0, out1,
                 sem_i0, sem_i1, sem_o0, sem_o1):
    wid = lax.axis_index("s") * NC + lax.axis_index("c")
    base = wid * PER_W
    iota4 = lax.iota(jnp.int32, L) * 4
    idx0 = iota4
    idx1 = iota4 + 1
    idx2 = iota4 + 2
    idx3 = iota4 + 3
    zero = jnp.zeros((L,), jnp.float32)

    def compute(src, dst):
        @plsc.parallel_loop(0, NIT, step=1, unroll=4)
        def _body(i):
            b = i * ELEMS_PER_IT
            s = src.at[pl.ds(b, ELEMS_PER_IT)]
            d = dst.at[pl.ds(b, ELEMS_PER_IT)]
            x0 = plsc.load_gather(s, [idx0])
            x1 = plsc.load_gather(s, [idx1])
            x2 = plsc.load_gather(s, [idx2])
            x3 = plsc.load_gather(s, [idx3])
            m = jnp.uint32(0x7FFFFFFF)
            key0 = lax.bitcast_convert_type(x0, jnp.uint32) & m
            key1 = (lax.bitcast_convert_type(x1, jnp.uint32) & m) + 1
            key2 = (lax.bitcast_convert_type(x2, jnp.uint32) & m) + 2
            key3 = (lax.bitcast_convert_type(x3, jnp.uint32) & m) + 3
            hi01 = jnp.maximum(key0, key1)
            lo01 = jnp.minimum(key0, key1)
            hi23 = jnp.maximum(key2, key3)
            lo23 = jnp.minimum(key2, key3)
            second = jnp.maximum(jnp.minimum(hi01, hi23),
                                 jnp.maximum(lo01, lo23))
            k0 = key0 >= second
            k1 = key1 >= second
            k2 = key2 >= second
            k3 = key3 >= second
            plsc.store_scatter(d, [idx0], jnp.where(k0, x0, zero))
            plsc.store_scatter(d, [idx1], jnp.where(k1, x1, zero))
            plsc.store_scatter(d, [idx2], jnp.where(k2, x2, zero))
            plsc.store_scatter(d, [idx3], jnp.where(k3, x3, zero))

    def halfstep(g, inb, outb, sem_i, sem_o):
        off = base + g * CHUNK
        pltpu.make_async_copy(x_hbm.at[pl.ds(off, CHUNK)], inb, sem_i).wait()
        @pl.when(g >= 2)
        def _():
            pltpu.make_async_copy(
                outb, out_hbm.at[pl.ds(off, CHUNK)], sem_o).wait()

        compute(inb, outb)
        pltpu.async_copy(outb, out_hbm.at[pl.ds(off, CHUNK)], sem_o)

        @pl.when(g + 2 < NSTEPS)
        def _():
            off2 = off + 2 * CHUNK
            pltpu.async_copy(x_hbm.at[pl.ds(off2, CHUNK)], inb, sem_i)

    pltpu.async_copy(x_hbm.at[pl.ds(base, CHUNK)], in0, sem_i0)
    pltpu.async_copy(x_hbm.at[pl.ds(base + CHUNK, CHUNK)], in1, sem_i1)

    def step(gg, carry):
        g = gg * 2
        halfstep(g, in0, out0, sem_i0, sem_o0)
        halfstep(g + 1, in1, out1, sem_i1, sem_o1)
        return carry

    lax.fori_loop(0, NSTEPS // 2, step, 0)

    pltpu.make_async_copy(
        out0, out_hbm.at[pl.ds(base, CHUNK)], sem_o0).wait()
    pltpu.make_async_copy(
        out1, out_hbm.at[pl.ds(base, CHUNK)], sem_o1).wait()


def kernel(inputs):
    flat = inputs.reshape(TOTAL)
    out = _sparsity_sc(flat)
    return out.reshape(inputs.shape)
